# Initial kernel scaffold; baseline (speedup 1.0000x reference)
import jax
import jax.numpy as jnp
from jax.experimental import pallas as pl


def kernel(x, detectors):
    raise NotImplementedError("write your pallas kernel here")


# speedup vs baseline: 2.9682x; 2.9682x over previous
import functools

import jax
import jax.numpy as jnp
from jax import lax
from jax.experimental import pallas as pl
from jax.experimental.pallas import tpu as pltpu
from jax.experimental.pallas import tpu_sc as plsc

B = 32
N = 32768
D = 8192
K = 16
NC = 2
NS = 16
CHUNK = 512


def _body(x_hbm, det_hbm, out_hbm, xrow, stat, dbuf):
    wid = lax.axis_index("s") * NC + lax.axis_index("c")

    pltpu.sync_copy(x_hbm.at[wid], xrow)

    def zero_step(i, _):
        stat[pl.ds(i * K, K)] = jnp.zeros((K,), jnp.int32)
        return 0
    lax.fori_loop(0, N // K, zero_step, 0)

    ones = jnp.ones((K,), jnp.int32)

    def chunk_step(cidx, _):
        pltpu.sync_copy(det_hbm.at[pl.ds(cidx * CHUNK, CHUNK)], dbuf)

        def det_step(j, _):
            ids = dbuf[j]
            vals = plsc.load_gather(xrow, [ids])
            m = jnp.max(vals)
            is_max = vals == m
            csum = jnp.cumsum(is_max.astype(jnp.int32), axis=0)
            loser = jnp.logical_not(jnp.logical_and(is_max, csum == 1))
            plsc.addupdate_scatter(stat, [ids], ones, mask=loser)
            return 0

        lax.fori_loop(0, CHUNK, det_step, 0)
        return 0

    lax.fori_loop(0, D // CHUNK, chunk_step, 0)

    def fin_step(i, _):
        s = stat[pl.ds(i * K, K)]
        xrow[pl.ds(i * K, K)] = jnp.where(s == 0, 1.0, 0.0).astype(jnp.float32)
        return 0
    lax.fori_loop(0, N // K, fin_step, 0)

    pltpu.sync_copy(xrow, out_hbm.at[wid])


@jax.jit
def kernel(x, detectors):
    run = pl.kernel(
        _body,
        out_type=jax.ShapeDtypeStruct((B, N), jnp.float32),
        mesh=plsc.VectorSubcoreMesh(
            core_axis_name="c", subcore_axis_name="s",
            num_cores=NC, num_subcores=NS,
        ),
        compiler_params=pltpu.CompilerParams(needs_layout_passes=False),
        scratch_types=[
            pltpu.VMEM((N,), jnp.float32),
            pltpu.VMEM((N,), jnp.int32),
            pltpu.VMEM((CHUNK, K), jnp.int32),
        ],
    )
    return run(x, detectors)


# speedup vs baseline: 14.3289x; 4.8274x over previous
import functools

import jax
import jax.numpy as jnp
from jax import lax
from jax.experimental import pallas as pl
from jax.experimental.pallas import tpu as pltpu
from jax.experimental.pallas import tpu_sc as plsc

B = 32
N = 32768
D = 8192
K = 16
NC = 2
NS = 16
GCH = 1024


def _tree_reduce(op, xs):
    xs = list(xs)
    while len(xs) > 1:
        nxt = [op(xs[i], xs[i + 1]) for i in range(0, len(xs) - 1, 2)]
        if len(xs) % 2:
            nxt.append(xs[-1])
        xs = nxt
    return xs[0]


def _body(x_hbm, det_hbm, out_hbm, xrow, stat, dbuf):
    wid = lax.axis_index("s") * NC + lax.axis_index("c")

    pltpu.sync_copy(x_hbm.at[wid], xrow)

    zeros = jnp.zeros((K,), jnp.int32)

    def zero_step(i, _):
        for u in range(8):
            stat[pl.ds((i * 8 + u) * K, K)] = zeros
        return 0
    lax.fori_loop(0, N // K // 8, zero_step, 0)

    ones = jnp.ones((K,), jnp.int32)

    def chunk_step(cidx, _):
        pltpu.sync_copy(det_hbm.at[:, pl.ds(cidx * GCH, GCH)], dbuf)

        def group_step(g, _):
            base = g * K
            ids = [dbuf[j, pl.ds(base, K)] for j in range(K)]
            vals = [plsc.load_gather(xrow, [ids[j]]) for j in range(K)]
            m = _tree_reduce(jnp.maximum, vals)
            wsel = [jnp.where(vals[j] == m, j, K) for j in range(K)]
            wmin = _tree_reduce(jnp.minimum, wsel)
            for j in range(K):
                plsc.addupdate_scatter(stat, [ids[j]], ones, mask=wmin != j)
            return 0

        lax.fori_loop(0, GCH // K, group_step, 0)
        return 0

    lax.fori_loop(0, D // GCH, chunk_step, 0)

    def fin_step(i, _):
        for u in range(8):
            s = stat[pl.ds((i * 8 + u) * K, K)]
            xrow[pl.ds((i * 8 + u) * K, K)] = jnp.where(s == 0, 1.0, 0.0)
        return 0
    lax.fori_loop(0, N // K // 8, fin_step, 0)

    pltpu.sync_copy(xrow, out_hbm.at[wid])


@jax.jit
def kernel(x, detectors):
    run = pl.kernel(
        _body,
        out_type=jax.ShapeDtypeStruct((B, N), jnp.float32),
        mesh=plsc.VectorSubcoreMesh(
            core_axis_name="c", subcore_axis_name="s",
            num_cores=NC, num_subcores=NS,
        ),
        compiler_params=pltpu.CompilerParams(needs_layout_passes=False),
        scratch_types=[
            pltpu.VMEM((N,), jnp.float32),
            pltpu.VMEM((N,), jnp.int32),
            pltpu.VMEM((K, GCH), jnp.int32),
        ],
    )
    det_t = detectors.T
    return run(x, det_t)
